# Initial kernel scaffold; baseline (speedup 1.0000x reference)
import jax
import jax.numpy as jnp
from jax.experimental import pallas as pl


def kernel(codes, codebooks):
    raise NotImplementedError("write your pallas kernel here")


# speedup vs baseline: 1.0989x; 1.0989x over previous
import functools

import jax
import jax.numpy as jnp
from jax import lax
from jax.experimental import pallas as pl
from jax.experimental.pallas import tpu as pltpu, tpu_sc as plsc

B, N_CB, T = 16, 8, 2048
K, D = 1024, 128

NC, NS = 2, 16
NW = NC * NS
TC = 256
PAIRS = B * N_CB
PAIRS_PER_W = PAIRS // NW
CHUNKS_PER_PAIR = T // TC
CHUNKS_PER_W = PAIRS_PER_W * CHUNKS_PER_PAIR
IDX_ROWS = TC // 128


def _body(idx_hbm, table_hbm, out_hbm, idx_v, rows_v, trans_v, sem):
    wid = lax.axis_index("s") * NC + lax.axis_index("c")
    iota16 = lax.iota(jnp.int32, 16)

    def chunk(ci, _):
        gchunk = wid * CHUNKS_PER_W + ci
        pair = gchunk // CHUNKS_PER_PAIR
        tchunk = gchunk % CHUNKS_PER_PAIR
        b = pair // N_CB
        i = pair % N_CB
        t0 = tchunk * TC

        pltpu.sync_copy(
            idx_hbm.at[b, i, pl.ds(tchunk * IDX_ROWS, IDX_ROWS), :], idx_v)

        for j in range(IDX_ROWS):
            pltpu.async_copy(
                table_hbm.at[idx_v.at[j]],
                rows_v.at[pl.ds(j * 128, 128)],
                sem,
            ).wait()

        def trow(j, _):
            col = jnp.broadcast_to(j, (16,)).astype(jnp.int32)
            for db in range(D // 16):
                v = rows_v[j, pl.ds(db * 16, 16)]
                plsc.store_scatter(trans_v, [db * 16 + iota16, col], v)
            return 0

        lax.fori_loop(0, TC, trow, 0)

        pltpu.sync_copy(
            trans_v, out_hbm.at[b, pl.ds(i * D, D), pl.ds(t0, TC)])
        return 0

    lax.fori_loop(0, CHUNKS_PER_W, chunk, 0)


@jax.jit
def _decode(idx, table):
    mesh = plsc.VectorSubcoreMesh(core_axis_name="c", subcore_axis_name="s")
    return pl.kernel(
        _body,
        out_type=jax.ShapeDtypeStruct((B, N_CB * D, T), jnp.float32),
        mesh=mesh,
        scratch_types=[
            pltpu.VMEM((IDX_ROWS, 128), jnp.int32),
            pltpu.VMEM((TC, D), jnp.float32),
            pltpu.VMEM((D, TC), jnp.float32),
            pltpu.SemaphoreType.DMA,
        ],
        compiler_params=pltpu.CompilerParams(
            use_tc_tiling_on_sc=False, needs_layout_passes=False),
    )(idx, table)


def kernel(codes, codebooks):
    idx = codes.astype(jnp.int32) + (jnp.arange(N_CB, dtype=jnp.int32) * K)[
        None, :, None]
    idx = idx.reshape(B, N_CB, T // 128, 128)
    table = codebooks.reshape(N_CB * K, D)
    return _decode(idx, table)


# speedup vs baseline: 1.3019x; 1.1847x over previous
import functools

import jax
import jax.numpy as jnp
from jax import lax
from jax.experimental import pallas as pl
from jax.experimental.pallas import tpu as pltpu, tpu_sc as plsc

B, N_CB, T = 16, 8, 2048
K, D = 1024, 128

NC, NS = 2, 16
NW = NC * NS
TC = 128
PAIRS = B * N_CB
PAIRS_PER_W = PAIRS // NW
CHUNKS_PER_PAIR = T // TC
NCHUNK = PAIRS_PER_W * CHUNKS_PER_PAIR


def _body(idx_hbm, table_hbm, out_hbm, idx_v, rows_v, trans_v,
          gsem0, gsem1, osem0, osem1):
    wid = lax.axis_index("s") * NC + lax.axis_index("c")
    iota16 = lax.iota(jnp.int32, 16)
    gsems = (gsem0, gsem1)
    osems = (osem0, osem1)

    pltpu.sync_copy(idx_hbm.at[pl.ds(wid * PAIRS_PER_W, PAIRS_PER_W)], idx_v)

    def out_slice(c):
        pair = wid * PAIRS_PER_W + c // CHUNKS_PER_PAIR
        t0 = (c % CHUNKS_PER_PAIR) * TC
        b = pair // N_CB
        i = pair % N_CB
        return out_hbm.at[b, pl.ds(i * D, D), pl.ds(t0, TC)]

    def fire_gather(c, buf):
        pltpu.async_copy(
            table_hbm.at[idx_v.at[c // CHUNKS_PER_PAIR,
                                  c % CHUNKS_PER_PAIR]],
            rows_v.at[buf], gsems[buf])

    def transpose(buf):
        rows = rows_v.at[buf]
        trans = trans_v.at[buf]

        def trow(j, _):
            col = jnp.broadcast_to(j, (16,)).astype(jnp.int32)
            for db in range(D // 16):
                v = rows[j, pl.ds(db * 16, 16)]
                plsc.store_scatter(trans, [db * 16 + iota16, col], v)
            return 0

        lax.fori_loop(0, TC, trow, 0, unroll=2)

    def wait_gather(c, buf):
        pltpu.make_async_copy(
            table_hbm.at[idx_v.at[0, 0]], rows_v.at[buf], gsems[buf]).wait()

    def fire_out(c, buf):
        pltpu.async_copy(trans_v.at[buf], out_slice(c), osems[buf])

    def wait_out(c, buf):
        pltpu.make_async_copy(trans_v.at[buf], out_slice(c), osems[buf]).wait()

    def step(c, buf, first, last):
        wait_gather(c, buf)
        if not first:
            wait_out(c - 2, buf)
        transpose(buf)
        fire_out(c, buf)
        if not last:
            fire_gather(c + 2, buf)

    fire_gather(0, 0)
    fire_gather(1, 1)
    step(0, 0, True, False)
    step(1, 1, True, False)

    def steady(g, _):
        c0 = 2 * g
        step(c0, 0, False, False)
        step(c0 + 1, 1, False, False)
        return 0

    lax.fori_loop(1, NCHUNK // 2 - 1, steady, 0)

    step(NCHUNK - 2, 0, False, True)
    step(NCHUNK - 1, 1, False, True)
    wait_out(NCHUNK - 2, 0)
    wait_out(NCHUNK - 1, 1)


@jax.jit
def _decode(idx, table):
    mesh = plsc.VectorSubcoreMesh(core_axis_name="c", subcore_axis_name="s")
    return pl.kernel(
        _body,
        out_type=jax.ShapeDtypeStruct((B, N_CB * D, T), jnp.float32),
        mesh=mesh,
        scratch_types=[
            pltpu.VMEM((PAIRS_PER_W, CHUNKS_PER_PAIR, TC), jnp.int32),
            pltpu.VMEM((2, TC, D), jnp.float32),
            pltpu.VMEM((2, D, TC), jnp.float32),
            pltpu.SemaphoreType.DMA,
            pltpu.SemaphoreType.DMA,
            pltpu.SemaphoreType.DMA,
            pltpu.SemaphoreType.DMA,
        ],
        compiler_params=pltpu.CompilerParams(
            use_tc_tiling_on_sc=False, needs_layout_passes=False),
    )(idx, table)


def kernel(codes, codebooks):
    idx = codes.astype(jnp.int32) + (jnp.arange(N_CB, dtype=jnp.int32) * K)[
        None, :, None]
    idx = idx.reshape(PAIRS, CHUNKS_PER_PAIR, TC)
    table = codebooks.reshape(N_CB * K, D)
    return _decode(idx, table)
